# baseline (speedup 1.0000x reference)
import functools

import jax
import jax.numpy as jnp
from jax import lax
from jax.experimental import pallas as pl
from jax.experimental.pallas import tpu as pltpu
from jax.experimental.pallas import tpu_sc as plsc

_BM = 1000

_NC, _NS = 2, 16
_NW = _NC * _NS
_SE = 1024
_NB = 26
_NBH = _NB // 2
_PW = _SE * _NB
_EP = _NW * _PW
_CAP = _NBH * _SE + 2 * 128
_CHR = 3456
_NCH = 15
_RT = _CHR * _NCH
_B2 = 128
_CROWS = _CAP // _B2
_NSL = _CHR // _B2


def _pad_rows(a, np_):
    return jnp.pad(a, ((0, np_ - a.shape[0]),) + ((0, 0),) * (a.ndim - 1))


def _d1_body(x_ref, w_ref, ats_ref, atd_ref, h_ref, as_ref, ad_ref):
    h = jnp.dot(x_ref[...], w_ref[...], preferred_element_type=jnp.float32)
    h_ref[...] = h
    as_ref[...] = jnp.sum(h * ats_ref[...], axis=1, keepdims=True)
    ad_ref[...] = jnp.sum(h * atd_ref[...], axis=1, keepdims=True)


def _dense1(xp, wp, att_s, att_d):
    np_, kp = xp.shape
    return pl.pallas_call(
        _d1_body,
        grid=(np_ // _BM,),
        in_specs=[
            pl.BlockSpec((_BM, kp), lambda i: (i, 0)),
            pl.BlockSpec((kp, 128), lambda i: (0, 0)),
            pl.BlockSpec((1, 128), lambda i: (0, 0)),
            pl.BlockSpec((1, 128), lambda i: (0, 0)),
        ],
        out_specs=[
            pl.BlockSpec((_BM, 128), lambda i: (i, 0)),
            pl.BlockSpec((_BM, 1), lambda i: (i, 0)),
            pl.BlockSpec((_BM, 1), lambda i: (i, 0)),
        ],
        out_shape=[
            jax.ShapeDtypeStruct((np_, 128), jnp.float32),
            jax.ShapeDtypeStruct((np_, 1), jnp.float32),
            jax.ShapeDtypeStruct((np_, 1), jnp.float32),
        ],
    )(xp, wp, att_s.reshape(1, 128), att_d.reshape(1, 128))


def _d2_body(acc0_ref, acc1_ref, den0_ref, den1_ref, b_ref, w_ref, ats_ref,
             atd_ref, h_ref, as_ref, ad_ref):
    acc = acc0_ref[...] + acc1_ref[...]
    den = den0_ref[...] + den1_ref[...]
    x2 = jnp.maximum(acc / (den + 1e-16) + b_ref[...], 0.0)
    h = jnp.dot(x2, w_ref[...], preferred_element_type=jnp.float32)
    h_ref[...] = h
    as_ref[...] = jnp.sum(h * ats_ref[...], axis=1, keepdims=True)
    ad_ref[...] = jnp.sum(h * atd_ref[...], axis=1, keepdims=True)


def _dense2(acc0, acc1, den0, den1, b, w, att_s, att_d):
    np_ = acc0.shape[0]
    return pl.pallas_call(
        _d2_body,
        grid=(np_ // _BM,),
        in_specs=[
            pl.BlockSpec((_BM, 128), lambda i: (i, 0)),
            pl.BlockSpec((_BM, 128), lambda i: (i, 0)),
            pl.BlockSpec((_BM, 1), lambda i: (i, 0)),
            pl.BlockSpec((_BM, 1), lambda i: (i, 0)),
            pl.BlockSpec((1, 128), lambda i: (0, 0)),
            pl.BlockSpec((128, 128), lambda i: (0, 0)),
            pl.BlockSpec((1, 128), lambda i: (0, 0)),
            pl.BlockSpec((1, 128), lambda i: (0, 0)),
        ],
        out_specs=[
            pl.BlockSpec((_BM, 128), lambda i: (i, 0)),
            pl.BlockSpec((_BM, 1), lambda i: (i, 0)),
            pl.BlockSpec((_BM, 1), lambda i: (i, 0)),
        ],
        out_shape=[
            jax.ShapeDtypeStruct((np_, 128), jnp.float32),
            jax.ShapeDtypeStruct((np_, 1), jnp.float32),
            jax.ShapeDtypeStruct((np_, 1), jnp.float32),
        ],
    )(acc0, acc1, den0, den1, b.reshape(1, 128), w, att_s.reshape(1, 128),
      att_d.reshape(1, 128))


def _d3_body(acc0_ref, acc1_ref, den0_ref, den1_ref, b_ref, wbT_ref, bb_ref,
             wwT_ref, mask_ref, bout_ref, g_ref):
    acc = acc0_ref[...] + acc1_ref[...]
    den = den0_ref[...] + den1_ref[...]
    x3 = jnp.maximum(acc / (den + 1e-16) + b_ref[...], 0.0)
    bout = jnp.sum(x3 * wbT_ref[...], axis=1, keepdims=True) + bb_ref[...]
    bout_ref[...] = bout * mask_ref[...]
    g_ref[...] = jnp.sum(x3 * wwT_ref[...], axis=1, keepdims=True)


def _dense3(acc0, acc1, den0, den1, b2, wb, bb, ww, mask):
    np_ = acc0.shape[0]
    return pl.pallas_call(
        _d3_body,
        grid=(np_ // _BM,),
        in_specs=[
            pl.BlockSpec((_BM, 128), lambda i: (i, 0)),
            pl.BlockSpec((_BM, 128), lambda i: (i, 0)),
            pl.BlockSpec((_BM, 1), lambda i: (i, 0)),
            pl.BlockSpec((_BM, 1), lambda i: (i, 0)),
            pl.BlockSpec((1, 128), lambda i: (0, 0)),
            pl.BlockSpec((1, 128), lambda i: (0, 0)),
            pl.BlockSpec((1, 1), lambda i: (0, 0)),
            pl.BlockSpec((1, 128), lambda i: (0, 0)),
            pl.BlockSpec((_BM, 1), lambda i: (i, 0)),
        ],
        out_specs=[
            pl.BlockSpec((_BM, 1), lambda i: (i, 0)),
            pl.BlockSpec((_BM, 1), lambda i: (i, 0)),
        ],
        out_shape=[
            jax.ShapeDtypeStruct((np_, 1), jnp.float32),
            jax.ShapeDtypeStruct((np_, 1), jnp.float32),
        ],
    )(acc0, acc1, den0, den1, b2.reshape(1, 128), wb.reshape(1, 128),
      bb.reshape(1, 1), ww.reshape(1, 128), mask)


def _mesh():
    return plsc.VectorSubcoreMesh(core_axis_name="c", subcore_axis_name="s",
                                  num_cores=_NC, num_subcores=_NS)


_SC_PARAMS = pltpu.CompilerParams(needs_layout_passes=False)


@functools.lru_cache(maxsize=None)
def _make_ea(np_, e_tot):
    def body(asrc, adst, srcp, dstp, eap, c16, exo, ts, td, sv, dv, ev, ov, cv):
        wid = lax.axis_index("c") * _NS + lax.axis_index("s")
        pltpu.sync_copy(asrc, ts)
        pltpu.sync_copy(adst, td)
        pltpu.sync_copy(c16, cv)
        c = cv[...]
        i16 = lax.iota(jnp.int32, 16)

        def jb_body(jb, carry):
            base = pl.multiple_of(wid * _PW + jb * _SE, 8)
            pltpu.sync_copy(srcp.at[pl.ds(base, _SE)], sv)
            pltpu.sync_copy(dstp.at[pl.ds(base, _SE)], dv)
            pltpu.sync_copy(eap.at[pl.ds(base, _SE)], ev)

            def ii_body(ii, carry2):
                lidx = ii * 16 + i16
                s = plsc.load_gather(sv, [lidx])
                d = plsc.load_gather(dv, [lidx])
                e_ = plsc.load_gather(ev, [lidx])
                a = (plsc.load_gather(ts, [s]) + plsc.load_gather(td, [d])
                     + e_ * c)
                a = jnp.maximum(a, 0.2 * a)
                exv = jnp.exp(a)
                exv = jnp.where(base + lidx < e_tot, exv, 0.0)
                plsc.store_scatter(ov, [lidx], exv)
                return carry2

            lax.fori_loop(0, _SE // 16, ii_body, 0)
            pltpu.sync_copy(ov, exo.at[pl.ds(base, _SE)])
            return carry

        lax.fori_loop(0, _NB, jb_body, 0)

    return pl.kernel(
        body,
        out_type=jax.ShapeDtypeStruct((_EP,), jnp.float32),
        mesh=_mesh(),
        compiler_params=_SC_PARAMS,
        scratch_types=[
            pltpu.VMEM((np_,), jnp.float32),
            pltpu.VMEM((np_,), jnp.float32),
            pltpu.VMEM((_SE,), jnp.int32),
            pltpu.VMEM((_SE,), jnp.int32),
            pltpu.VMEM((_SE,), jnp.float32),
            pltpu.VMEM((_SE,), jnp.float32),
            pltpu.VMEM((16,), jnp.float32),
        ],
    )


@functools.lru_cache(maxsize=None)
def _make_ew(np_, e_tot):
    def body(g, srcp, dstp, bw16, wo, tg, sv, dv, ov, cv):
        wid = lax.axis_index("c") * _NS + lax.axis_index("s")
        pltpu.sync_copy(g, tg)
        pltpu.sync_copy(bw16, cv)
        bwv = cv[...]
        i16 = lax.iota(jnp.int32, 16)

        def jb_body(jb, carry):
            base = pl.multiple_of(wid * _PW + jb * _SE, 8)
            pltpu.sync_copy(srcp.at[pl.ds(base, _SE)], sv)
            pltpu.sync_copy(dstp.at[pl.ds(base, _SE)], dv)

            def ii_body(ii, carry2):
                lidx = ii * 16 + i16
                s = plsc.load_gather(sv, [lidx])
                d = plsc.load_gather(dv, [lidx])
                w = (plsc.load_gather(tg, [s])
                     + plsc.load_gather(tg, [d])) * 0.5 + bwv
                plsc.store_scatter(ov, [lidx], w)
                return carry2

            lax.fori_loop(0, _SE // 16, ii_body, 0)
            pltpu.sync_copy(ov, wo.at[pl.ds(base, _SE)])
            return carry

        lax.fori_loop(0, _NB, jb_body, 0)

    return pl.kernel(
        body,
        out_type=jax.ShapeDtypeStruct((_EP,), jnp.float32),
        mesh=_mesh(),
        compiler_params=_SC_PARAMS,
        scratch_types=[
            pltpu.VMEM((np_,), jnp.float32),
            pltpu.VMEM((_SE,), jnp.int32),
            pltpu.VMEM((_SE,), jnp.int32),
            pltpu.VMEM((_SE,), jnp.float32),
            pltpu.VMEM((16,), jnp.float32),
        ],
    )


@functools.lru_cache(maxsize=None)
def _make_eb(np_):
    def body(h, srcp, dstp, xin, zacc, zden, acco, deno,
             sv, dv, ev, cs, cd2, cex, rb0, rb1, gs0, gs1, ss0, ss1,
             acc_sh, den_sh):
        cid = lax.axis_index("c")
        sid = lax.axis_index("s")
        wid = cid * _NS + sid
        i16 = lax.iota(jnp.int32, 16)
        z16i = jnp.zeros((16,), jnp.int32)
        z16f = jnp.zeros((16,), jnp.float32)

        nsl_mine = (_NSL - 1) // _NS + 1

        def ch_body(ci, carry):
            lo = ci * _CHR

            def z_body(t, carry0):
                j = sid + t * _NS

                @pl.when(j < _NSL)
                def _():
                    j0 = pl.multiple_of(j * _B2, 8)
                    pltpu.sync_copy(zacc, acc_sh.at[pl.ds(j0, _B2)])
                    pltpu.sync_copy(zden, den_sh.at[pl.ds(j0, _B2)])
                return carry0

            lax.fori_loop(0, nsl_mine, z_body, 0)
            plsc.subcore_barrier()

            def half_body(hf, carryh):
                def jb_body(jb, k):
                    base = pl.multiple_of(
                        wid * _PW + (hf * _NBH + jb) * _SE, 8)
                    pltpu.sync_copy(srcp.at[pl.ds(base, _SE)], sv)
                    pltpu.sync_copy(dstp.at[pl.ds(base, _SE)], dv)
                    pltpu.sync_copy(xin.at[pl.ds(base, _SE)], ev)

                    for ii in range(_SE // 16):
                        sl = pl.ds(ii * 16, 16)
                        s = sv[sl]
                        d = dv[sl]
                        x = ev[sl]
                        m = (d >= lo) & (d < lo + _CHR) & (x != 0.0)
                        mi = m.astype(jnp.int32)
                        pos = k + plsc.cumsum(mi) - 1
                        pr, pc = pos // _B2, pos % _B2
                        plsc.store_scatter(cs, [pr, pc], s, mask=m)
                        plsc.store_scatter(cex, [pos], x, mask=m)
                        plsc.store_scatter(cd2, [pr, pc], d - lo, mask=m)
                        k = k + plsc.all_reduce_population_count(m)[0]
                    return k

                k = lax.fori_loop(0, _NBH, jb_body, 0)

                kpad = ((k + 2 * _B2 - 1) // (2 * _B2)) * (2 * _B2)
                for t in range(2 * _B2 // 16):
                    idx = k + t * 16 + i16
                    m = idx < kpad
                    ir, ic = idx // _B2, idx % _B2
                    plsc.store_scatter(cs, [ir, ic], z16i, mask=m)
                    plsc.store_scatter(cex, [idx], z16f, mask=m)
                    plsc.store_scatter(cd2, [ir, ic], z16i, mask=m)

                nblk = kpad // _B2

                def scale_rows(rb, bo):
                    def r_body(r4, carry3):
                        for u in range(4):
                            r = r4 * 4 + u
                            exv = plsc.load_gather(
                                cex, [jnp.full((16,), bo + r, jnp.int32)])
                            rfull = jnp.full((16,), r, jnp.int32)
                            for q in range(8):
                                cidx = q * 16 + i16
                                v = plsc.load_gather(rb, [rfull, cidx])
                                plsc.store_scatter(rb, [rfull, cidx],
                                                   v * exv)
                        return carry3

                    lax.fori_loop(0, _B2 // 4, r_body, 0)

                def do_block(b, rb, gsem, ssem):
                    pltpu.make_async_copy(h.at[cs.at[0]], rb, gsem).wait()
                    scale_rows(rb, b * _B2)
                    pltpu.async_copy(rb, acc_sh.at[cd2.at[b]], ssem,
                                     add=True)
                    pltpu.sync_copy(
                        cex.at[pl.ds(pl.multiple_of(b * _B2, 8), _B2)],
                        den_sh.at[cd2.at[b]], add=True)

                    @pl.when(b + 2 < nblk)
                    def _():
                        pltpu.make_async_copy(rb, acc_sh.at[cd2.at[0]],
                                              ssem).wait()
                        pltpu.async_copy(h.at[cs.at[b + 2]], rb, gsem)

                @pl.when(nblk > 0)
                def _():
                    pltpu.async_copy(h.at[cs.at[0]], rb0, gs0)
                    pltpu.async_copy(h.at[cs.at[1]], rb1, gs1)

                def p_body(p, carry2):
                    do_block(p * 2, rb0, gs0, ss0)
                    do_block(p * 2 + 1, rb1, gs1, ss1)
                    return carry2

                lax.fori_loop(0, nblk // 2, p_body, 0)

                @pl.when(nblk > 0)
                def _():
                    pltpu.make_async_copy(rb0, acc_sh.at[cd2.at[0]],
                                          ss0).wait()
                    pltpu.make_async_copy(rb1, acc_sh.at[cd2.at[0]],
                                          ss1).wait()
                return carryh

            lax.fori_loop(0, 2, half_body, 0)
            plsc.subcore_barrier()

            def o_body(t, carry0):
                j = sid + t * _NS

                @pl.when(j < _NSL)
                def _():
                    j0 = pl.multiple_of(j * _B2, 8)
                    row0 = pl.multiple_of(
                        cid * _RT + ci * _CHR + j * _B2, 8)
                    pltpu.sync_copy(acc_sh.at[pl.ds(j0, _B2)],
                                    acco.at[pl.ds(row0, _B2)])
                    pltpu.sync_copy(den_sh.at[pl.ds(j0, _B2)],
                                    deno.at[pl.ds(row0, _B2)])
                return carry0

            lax.fori_loop(0, nsl_mine, o_body, 0)
            plsc.subcore_barrier()
            return carry

        lax.fori_loop(0, _NCH, ch_body, 0)

    return pl.kernel(
        body,
        out_type=[
            jax.ShapeDtypeStruct((_NC * _RT, 128), jnp.float32),
            jax.ShapeDtypeStruct((_NC * _RT,), jnp.float32),
        ],
        mesh=_mesh(),
        compiler_params=_SC_PARAMS,
        scratch_types=[
            pltpu.VMEM((_SE,), jnp.int32),
            pltpu.VMEM((_SE,), jnp.int32),
            pltpu.VMEM((_SE,), jnp.float32),
            pltpu.VMEM((_CROWS, _B2), jnp.int32),
            pltpu.VMEM((_CROWS, _B2), jnp.int32),
            pltpu.VMEM((_PW,), jnp.float32),
            pltpu.VMEM((_B2, 128), jnp.float32),
            pltpu.VMEM((_B2, 128), jnp.float32),
            pltpu.SemaphoreType.DMA,
            pltpu.SemaphoreType.DMA,
            pltpu.SemaphoreType.DMA,
            pltpu.SemaphoreType.DMA,
            pltpu.VMEM_SHARED((_CHR, 128), jnp.float32),
            pltpu.VMEM_SHARED((_CHR,), jnp.float32),
        ],
    )


def _edge_stage(h, a_src, a_dst, srcp, dstp, eap, c, np_, e_tot):
    ex = _make_ea(np_, e_tot)(a_src, a_dst, srcp, dstp, eap,
                              jnp.full((16,), c, jnp.float32))
    zacc = jnp.zeros((_B2, 128), jnp.float32)
    zden = jnp.zeros((_B2,), jnp.float32)
    acc, den = _make_eb(np_)(h, srcp, dstp, ex, zacc, zden)
    return acc, den


def kernel(x, edge_index, edge_attr, input_mask, W1, att_s1, att_d1, We1,
           att_e1, b1, W2, att_s2, att_d2, We2, att_e2, b2, Wb, bb, Ww, bw):
    n, d_in = x.shape
    e = edge_index.shape[1]
    np_ = n
    srcp = jnp.pad(edge_index[0], (0, _EP - e))
    dstp = jnp.pad(edge_index[1], (0, _EP - e))
    eap = jnp.pad(edge_attr[:, 0], (0, _EP - e))

    h1, as1, ad1 = _dense1(x, W1, att_s1, att_d1)
    c1 = jnp.dot(We1[0], att_e1)
    acc1, den1 = _edge_stage(h1, as1.reshape(-1), ad1.reshape(-1),
                             srcp, dstp, eap, c1, np_, e)
    h2, as2, ad2 = _dense2(acc1[:np_], acc1[_RT:_RT + np_],
                           den1[:np_, None], den1[_RT:_RT + np_, None],
                           b1, W2, att_s2, att_d2)
    c2 = jnp.dot(We2[0], att_e2)
    acc2, den2 = _edge_stage(h2, as2.reshape(-1), ad2.reshape(-1),
                             srcp, dstp, eap, c2, np_, e)
    bout, g = _dense3(acc2[:np_], acc2[_RT:_RT + np_],
                      den2[:np_, None], den2[_RT:_RT + np_, None],
                      b2, Wb, bb, Ww, input_mask)
    wout = _make_ew(np_, e)(g.reshape(-1), srcp, dstp,
                            jnp.full((16,), bw[0], jnp.float32))
    return (wout[:e, None], bout[:n])


# speedup vs baseline: 1.6593x; 1.6593x over previous
import functools

import jax
import jax.numpy as jnp
from jax import lax
from jax.experimental import pallas as pl
from jax.experimental.pallas import tpu as pltpu
from jax.experimental.pallas import tpu_sc as plsc

_BM = 1000

_NC, _NS = 2, 16
_NW = _NC * _NS
_SE = 2048
_NB = 13
_PW = _SE * _NB
_EP = _NW * _PW
_CAP = _PW
_CHR = 3456
_NCH = 15
_RT = _CHR * _NCH
_B2 = 128
_CROWS = _CAP // _B2
_NSL = _CHR // _B2


def _pad_rows(a, np_):
    return jnp.pad(a, ((0, np_ - a.shape[0]),) + ((0, 0),) * (a.ndim - 1))


def _d1_body(x_ref, w_ref, ats_ref, atd_ref, h_ref, as_ref, ad_ref):
    h = jnp.dot(x_ref[...], w_ref[...], preferred_element_type=jnp.float32)
    h_ref[...] = h
    as_ref[...] = jnp.sum(h * ats_ref[...], axis=1, keepdims=True)
    ad_ref[...] = jnp.sum(h * atd_ref[...], axis=1, keepdims=True)


def _dense1(xp, wp, att_s, att_d):
    np_, kp = xp.shape
    return pl.pallas_call(
        _d1_body,
        grid=(np_ // _BM,),
        in_specs=[
            pl.BlockSpec((_BM, kp), lambda i: (i, 0)),
            pl.BlockSpec((kp, 128), lambda i: (0, 0)),
            pl.BlockSpec((1, 128), lambda i: (0, 0)),
            pl.BlockSpec((1, 128), lambda i: (0, 0)),
        ],
        out_specs=[
            pl.BlockSpec((_BM, 128), lambda i: (i, 0)),
            pl.BlockSpec((_BM, 1), lambda i: (i, 0)),
            pl.BlockSpec((_BM, 1), lambda i: (i, 0)),
        ],
        out_shape=[
            jax.ShapeDtypeStruct((np_, 128), jnp.float32),
            jax.ShapeDtypeStruct((np_, 1), jnp.float32),
            jax.ShapeDtypeStruct((np_, 1), jnp.float32),
        ],
    )(xp, wp, att_s.reshape(1, 128), att_d.reshape(1, 128))


def _d2_body(acc0_ref, acc1_ref, den0_ref, den1_ref, b_ref, w_ref, ats_ref,
             atd_ref, h_ref, as_ref, ad_ref):
    acc = acc0_ref[...] + acc1_ref[...]
    den = den0_ref[...] + den1_ref[...]
    x2 = jnp.maximum(acc / (den + 1e-16) + b_ref[...], 0.0)
    h = jnp.dot(x2, w_ref[...], preferred_element_type=jnp.float32)
    h_ref[...] = h
    as_ref[...] = jnp.sum(h * ats_ref[...], axis=1, keepdims=True)
    ad_ref[...] = jnp.sum(h * atd_ref[...], axis=1, keepdims=True)


def _dense2(acc0, acc1, den0, den1, b, w, att_s, att_d):
    np_ = acc0.shape[0]
    return pl.pallas_call(
        _d2_body,
        grid=(np_ // _BM,),
        in_specs=[
            pl.BlockSpec((_BM, 128), lambda i: (i, 0)),
            pl.BlockSpec((_BM, 128), lambda i: (i, 0)),
            pl.BlockSpec((_BM, 1), lambda i: (i, 0)),
            pl.BlockSpec((_BM, 1), lambda i: (i, 0)),
            pl.BlockSpec((1, 128), lambda i: (0, 0)),
            pl.BlockSpec((128, 128), lambda i: (0, 0)),
            pl.BlockSpec((1, 128), lambda i: (0, 0)),
            pl.BlockSpec((1, 128), lambda i: (0, 0)),
        ],
        out_specs=[
            pl.BlockSpec((_BM, 128), lambda i: (i, 0)),
            pl.BlockSpec((_BM, 1), lambda i: (i, 0)),
            pl.BlockSpec((_BM, 1), lambda i: (i, 0)),
        ],
        out_shape=[
            jax.ShapeDtypeStruct((np_, 128), jnp.float32),
            jax.ShapeDtypeStruct((np_, 1), jnp.float32),
            jax.ShapeDtypeStruct((np_, 1), jnp.float32),
        ],
    )(acc0, acc1, den0, den1, b.reshape(1, 128), w, att_s.reshape(1, 128),
      att_d.reshape(1, 128))


def _d3_body(acc0_ref, acc1_ref, den0_ref, den1_ref, b_ref, wbT_ref, bb_ref,
             wwT_ref, mask_ref, bout_ref, g_ref):
    acc = acc0_ref[...] + acc1_ref[...]
    den = den0_ref[...] + den1_ref[...]
    x3 = jnp.maximum(acc / (den + 1e-16) + b_ref[...], 0.0)
    bout = jnp.sum(x3 * wbT_ref[...], axis=1, keepdims=True) + bb_ref[...]
    bout_ref[...] = bout * mask_ref[...]
    g_ref[...] = jnp.sum(x3 * wwT_ref[...], axis=1, keepdims=True)


def _dense3(acc0, acc1, den0, den1, b2, wb, bb, ww, mask):
    np_ = acc0.shape[0]
    return pl.pallas_call(
        _d3_body,
        grid=(np_ // _BM,),
        in_specs=[
            pl.BlockSpec((_BM, 128), lambda i: (i, 0)),
            pl.BlockSpec((_BM, 128), lambda i: (i, 0)),
            pl.BlockSpec((_BM, 1), lambda i: (i, 0)),
            pl.BlockSpec((_BM, 1), lambda i: (i, 0)),
            pl.BlockSpec((1, 128), lambda i: (0, 0)),
            pl.BlockSpec((1, 128), lambda i: (0, 0)),
            pl.BlockSpec((1, 1), lambda i: (0, 0)),
            pl.BlockSpec((1, 128), lambda i: (0, 0)),
            pl.BlockSpec((_BM, 1), lambda i: (i, 0)),
        ],
        out_specs=[
            pl.BlockSpec((_BM, 1), lambda i: (i, 0)),
            pl.BlockSpec((_BM, 1), lambda i: (i, 0)),
        ],
        out_shape=[
            jax.ShapeDtypeStruct((np_, 1), jnp.float32),
            jax.ShapeDtypeStruct((np_, 1), jnp.float32),
        ],
    )(acc0, acc1, den0, den1, b2.reshape(1, 128), wb.reshape(1, 128),
      bb.reshape(1, 1), ww.reshape(1, 128), mask)


def _mesh():
    return plsc.VectorSubcoreMesh(core_axis_name="c", subcore_axis_name="s",
                                  num_cores=_NC, num_subcores=_NS)


_SC_PARAMS = pltpu.CompilerParams(needs_layout_passes=False)


@functools.lru_cache(maxsize=None)
def _make_ea(np_, e_tot):
    def body(asrc, adst, srcp, dstp, eap, c16, exo, ts, td, sv, dv, ev, ov, cv):
        wid = lax.axis_index("c") * _NS + lax.axis_index("s")
        pltpu.sync_copy(asrc, ts)
        pltpu.sync_copy(adst, td)
        pltpu.sync_copy(c16, cv)
        c = cv[...]
        i16 = lax.iota(jnp.int32, 16)

        def jb_body(jb, carry):
            base = pl.multiple_of(wid * _PW + jb * _SE, 8)
            pltpu.sync_copy(srcp.at[pl.ds(base, _SE)], sv)
            pltpu.sync_copy(dstp.at[pl.ds(base, _SE)], dv)
            pltpu.sync_copy(eap.at[pl.ds(base, _SE)], ev)

            def ii_body(ii, carry2):
                lidx = ii * 16 + i16
                s = plsc.load_gather(sv, [lidx])
                d = plsc.load_gather(dv, [lidx])
                e_ = plsc.load_gather(ev, [lidx])
                a = (plsc.load_gather(ts, [s]) + plsc.load_gather(td, [d])
                     + e_ * c)
                a = jnp.maximum(a, 0.2 * a)
                exv = jnp.exp(a)
                exv = jnp.where(base + lidx < e_tot, exv, 0.0)
                plsc.store_scatter(ov, [lidx], exv)
                return carry2

            lax.fori_loop(0, _SE // 16, ii_body, 0)
            pltpu.sync_copy(ov, exo.at[pl.ds(base, _SE)])
            return carry

        lax.fori_loop(0, _NB, jb_body, 0)

    return pl.kernel(
        body,
        out_type=jax.ShapeDtypeStruct((_EP,), jnp.float32),
        mesh=_mesh(),
        compiler_params=_SC_PARAMS,
        scratch_types=[
            pltpu.VMEM((np_,), jnp.float32),
            pltpu.VMEM((np_,), jnp.float32),
            pltpu.VMEM((_SE,), jnp.int32),
            pltpu.VMEM((_SE,), jnp.int32),
            pltpu.VMEM((_SE,), jnp.float32),
            pltpu.VMEM((_SE,), jnp.float32),
            pltpu.VMEM((16,), jnp.float32),
        ],
    )


@functools.lru_cache(maxsize=None)
def _make_ew(np_, e_tot):
    def body(g, srcp, dstp, bw16, wo, tg, sv, dv, ov, cv):
        wid = lax.axis_index("c") * _NS + lax.axis_index("s")
        pltpu.sync_copy(g, tg)
        pltpu.sync_copy(bw16, cv)
        bwv = cv[...]
        i16 = lax.iota(jnp.int32, 16)

        def jb_body(jb, carry):
            base = pl.multiple_of(wid * _PW + jb * _SE, 8)
            pltpu.sync_copy(srcp.at[pl.ds(base, _SE)], sv)
            pltpu.sync_copy(dstp.at[pl.ds(base, _SE)], dv)

            def ii_body(ii, carry2):
                lidx = ii * 16 + i16
                s = plsc.load_gather(sv, [lidx])
                d = plsc.load_gather(dv, [lidx])
                w = (plsc.load_gather(tg, [s])
                     + plsc.load_gather(tg, [d])) * 0.5 + bwv
                plsc.store_scatter(ov, [lidx], w)
                return carry2

            lax.fori_loop(0, _SE // 16, ii_body, 0)
            pltpu.sync_copy(ov, wo.at[pl.ds(base, _SE)])
            return carry

        lax.fori_loop(0, _NB, jb_body, 0)

    return pl.kernel(
        body,
        out_type=jax.ShapeDtypeStruct((_EP,), jnp.float32),
        mesh=_mesh(),
        compiler_params=_SC_PARAMS,
        scratch_types=[
            pltpu.VMEM((np_,), jnp.float32),
            pltpu.VMEM((_SE,), jnp.int32),
            pltpu.VMEM((_SE,), jnp.int32),
            pltpu.VMEM((_SE,), jnp.float32),
            pltpu.VMEM((16,), jnp.float32),
        ],
    )


@functools.lru_cache(maxsize=None)
def _make_eb(np_):
    def body(h, srcp, dstp, xin, zacc, zden, acco, deno,
             sv, dv, ev, cp, cex, csb, cdb, rb0, rb1, gs0, gs1,
             acc_sh, den_sh):
        cid = lax.axis_index("c")
        sid = lax.axis_index("s")
        wid = cid * _NS + sid
        i16 = lax.iota(jnp.int32, 16)
        z16i = jnp.zeros((16,), jnp.int32)
        z16f = jnp.zeros((16,), jnp.float32)

        nsl_mine = (_NSL - 1) // _NS + 1

        def ch_body(ci, carry):
            lo = ci * _CHR

            def z_body(t, carry0):
                j = sid + t * _NS

                @pl.when(j < _NSL)
                def _():
                    j0 = pl.multiple_of(j * _B2, 8)
                    pltpu.sync_copy(zacc, acc_sh.at[pl.ds(j0, _B2)])
                    pltpu.sync_copy(zden, den_sh.at[pl.ds(j0, _B2)])
                return carry0

            lax.fori_loop(0, nsl_mine, z_body, 0)
            plsc.subcore_barrier()

            def jb_body(jb, k):
                base = pl.multiple_of(wid * _PW + jb * _SE, 8)
                pltpu.sync_copy(srcp.at[pl.ds(base, _SE)], sv)
                pltpu.sync_copy(dstp.at[pl.ds(base, _SE)], dv)
                pltpu.sync_copy(xin.at[pl.ds(base, _SE)], ev)

                def ii_body(ii, k2):
                    lidx = ii * 16 + i16
                    s = plsc.load_gather(sv, [lidx])
                    d = plsc.load_gather(dv, [lidx])
                    x = plsc.load_gather(ev, [lidx])
                    m = (d >= lo) & (d < lo + _CHR) & (x != 0.0)
                    mi = m.astype(jnp.int32)
                    pos = k2 + plsc.cumsum(mi) - 1
                    pr, pc = pos // _B2, pos % _B2
                    plsc.store_scatter(cp, [pr, pc],
                                       s + ((d - lo) << 16), mask=m)
                    plsc.store_scatter(cex, [pos], x, mask=m)
                    return k2 + plsc.all_reduce_population_count(m)[0]

                return lax.fori_loop(0, _SE // 16, ii_body, k)

            k = lax.fori_loop(0, _NB, jb_body, 0)

            kpad = ((k + _B2 - 1) // _B2) * _B2
            for t in range(_B2 // 16):
                idx = k + t * 16 + i16
                m = idx < kpad
                ir, ic = idx // _B2, idx % _B2
                plsc.store_scatter(cp, [ir, ic], z16i, mask=m)
                plsc.store_scatter(cex, [idx], z16f, mask=m)

            nblk = kpad // _B2

            def unpack(b, par):
                pfull = jnp.full((16,), par, jnp.int32)
                bfull = jnp.full((16,), b, jnp.int32)
                for q in range(8):
                    cidx = q * 16 + i16
                    v = plsc.load_gather(cp, [bfull, cidx])
                    plsc.store_scatter(csb, [pfull, cidx], v & 0xFFFF)
                    plsc.store_scatter(cdb, [pfull, cidx], v >> 16)

            def scale_scatter(b, par, rb):
                bo = pl.multiple_of(b * _B2, 8)

                def r_body(r, carry3):
                    exv = plsc.load_gather(
                        cex, [jnp.full((16,), bo + r, jnp.int32)])
                    rfull = jnp.full((16,), r, jnp.int32)
                    for q in range(8):
                        cidx = q * 16 + i16
                        v = plsc.load_gather(rb, [rfull, cidx])
                        plsc.store_scatter(rb, [rfull, cidx], v * exv)
                    return carry3

                lax.fori_loop(0, _B2, r_body, 0)
                pltpu.sync_copy(rb, acc_sh.at[cdb.at[par]], add=True)
                pltpu.sync_copy(cex.at[pl.ds(bo, _B2)],
                                den_sh.at[cdb.at[par]], add=True)

            @pl.when(nblk > 0)
            def _():
                unpack(0, 0)
                pltpu.async_copy(h.at[csb.at[0]], rb0, gs0)

            def p_body(p, carry2):
                b0 = p * 2
                b1 = b0 + 1
                pltpu.make_async_copy(h.at[csb.at[0]], rb0, gs0).wait()

                @pl.when(b1 < nblk)
                def _():
                    unpack(b1, 1)
                    pltpu.async_copy(h.at[csb.at[1]], rb1, gs1)

                scale_scatter(b0, 0, rb0)

                @pl.when(b1 < nblk)
                def _():
                    pltpu.make_async_copy(h.at[csb.at[1]], rb1, gs1).wait()

                    @pl.when(b1 + 1 < nblk)
                    def _():
                        unpack(b1 + 1, 0)
                        pltpu.async_copy(h.at[csb.at[0]], rb0, gs0)

                    scale_scatter(b1, 1, rb1)
                return carry2

            lax.fori_loop(0, (nblk + 1) // 2, p_body, 0)
            plsc.subcore_barrier()

            def o_body(t, carry0):
                j = sid + t * _NS

                @pl.when(j < _NSL)
                def _():
                    j0 = pl.multiple_of(j * _B2, 8)
                    row0 = pl.multiple_of(
                        cid * _RT + ci * _CHR + j * _B2, 8)
                    pltpu.sync_copy(acc_sh.at[pl.ds(j0, _B2)],
                                    acco.at[pl.ds(row0, _B2)])
                    pltpu.sync_copy(den_sh.at[pl.ds(j0, _B2)],
                                    deno.at[pl.ds(row0, _B2)])
                return carry0

            lax.fori_loop(0, nsl_mine, o_body, 0)
            plsc.subcore_barrier()
            return carry

        lax.fori_loop(0, _NCH, ch_body, 0)

    return pl.kernel(
        body,
        out_type=[
            jax.ShapeDtypeStruct((_NC * _RT, 128), jnp.float32),
            jax.ShapeDtypeStruct((_NC * _RT,), jnp.float32),
        ],
        mesh=_mesh(),
        compiler_params=_SC_PARAMS,
        scratch_types=[
            pltpu.VMEM((_SE,), jnp.int32),
            pltpu.VMEM((_SE,), jnp.int32),
            pltpu.VMEM((_SE,), jnp.float32),
            pltpu.VMEM((_CROWS, _B2), jnp.int32),
            pltpu.VMEM((_CAP,), jnp.float32),
            pltpu.VMEM((2, _B2), jnp.int32),
            pltpu.VMEM((2, _B2), jnp.int32),
            pltpu.VMEM((_B2, 128), jnp.float32),
            pltpu.VMEM((_B2, 128), jnp.float32),
            pltpu.SemaphoreType.DMA,
            pltpu.SemaphoreType.DMA,
            pltpu.VMEM_SHARED((_CHR, 128), jnp.float32),
            pltpu.VMEM_SHARED((_CHR,), jnp.float32),
        ],
    )


def _edge_stage(h, a_src, a_dst, srcp, dstp, eap, c, np_, e_tot):
    ex = _make_ea(np_, e_tot)(a_src, a_dst, srcp, dstp, eap,
                              jnp.full((16,), c, jnp.float32))
    zacc = jnp.zeros((_B2, 128), jnp.float32)
    zden = jnp.zeros((_B2,), jnp.float32)
    acc, den = _make_eb(np_)(h, srcp, dstp, ex, zacc, zden)
    return acc, den


def kernel(x, edge_index, edge_attr, input_mask, W1, att_s1, att_d1, We1,
           att_e1, b1, W2, att_s2, att_d2, We2, att_e2, b2, Wb, bb, Ww, bw):
    n, d_in = x.shape
    e = edge_index.shape[1]
    np_ = n
    srcp = jnp.pad(edge_index[0], (0, _EP - e))
    dstp = jnp.pad(edge_index[1], (0, _EP - e))
    eap = jnp.pad(edge_attr[:, 0], (0, _EP - e))

    h1, as1, ad1 = _dense1(x, W1, att_s1, att_d1)
    c1 = jnp.dot(We1[0], att_e1)
    acc1, den1 = _edge_stage(h1, as1.reshape(-1), ad1.reshape(-1),
                             srcp, dstp, eap, c1, np_, e)
    h2, as2, ad2 = _dense2(acc1[:np_], acc1[_RT:_RT + np_],
                           den1[:np_, None], den1[_RT:_RT + np_, None],
                           b1, W2, att_s2, att_d2)
    c2 = jnp.dot(We2[0], att_e2)
    acc2, den2 = _edge_stage(h2, as2.reshape(-1), ad2.reshape(-1),
                             srcp, dstp, eap, c2, np_, e)
    bout, g = _dense3(acc2[:np_], acc2[_RT:_RT + np_],
                      den2[:np_, None], den2[_RT:_RT + np_, None],
                      b2, Wb, bb, Ww, input_mask)
    wout = _make_ew(np_, e)(g.reshape(-1), srcp, dstp,
                            jnp.full((16,), bw[0], jnp.float32))
    return (wout[:e, None], bout[:n])
